# baseline (speedup 1.0000x reference)
import functools

import jax
import jax.numpy as jnp
import numpy as np
from jax import lax
from jax.experimental import pallas as pl
from jax.experimental.pallas import tpu as pltpu
from jax.experimental.pallas import tpu_sc as plsc

N_VOCAB = 1000000
D_MODEL = 64
BATCH = 1024
SEQ_LEN = 200

_NW = 32
_TOTAL = BATCH * SEQ_LEN
_PER_W = _TOTAL // _NW
_CHUNK = 400
_NCHUNK = _PER_W // _CHUNK
_SUB = 80
_NSUB = _CHUNK // _SUB


def _pe_table():
    pos = np.arange(SEQ_LEN, dtype=np.float32)[:, None]
    div = np.exp(np.arange(0, D_MODEL, 2, dtype=np.float32)
                 * (-np.log(10000.0) / D_MODEL))
    pe = np.zeros((SEQ_LEN, D_MODEL), dtype=np.float32)
    pe[:, 0::2] = np.sin(pos * div)
    pe[:, 1::2] = np.cos(pos * div)
    return jnp.asarray(pe)


def _sc_embed(idx_flat, table, pe):
    mesh = plsc.VectorSubcoreMesh(core_axis_name="c", subcore_axis_name="s")

    @functools.partial(
        pl.kernel,
        out_type=jax.ShapeDtypeStruct((_TOTAL, D_MODEL), jnp.float32),
        mesh=mesh,
        scratch_types=[
            pltpu.VMEM((_PER_W,), jnp.int32),
            pltpu.VMEM((SEQ_LEN, D_MODEL), jnp.float32),
            pltpu.VMEM((_CHUNK, D_MODEL), jnp.float32),
            pltpu.SemaphoreType.DMA,
        ],
        compiler_params=pltpu.CompilerParams(use_tc_tiling_on_sc=False),
    )
    def k(table_hbm, idx_hbm, pe_hbm, out_hbm, idx_v, pe_v, buf, sem):
        wid = lax.axis_index("s") * 2 + lax.axis_index("c")
        base = wid * _PER_W
        pltpu.sync_copy(idx_hbm.at[pl.ds(base, _PER_W)], idx_v)
        pltpu.sync_copy(pe_hbm, pe_v)

        def chunk_body(g, carry):
            row0 = g * _CHUNK
            copies = []
            for j in range(_NSUB):
                copies.append(pltpu.async_copy(
                    table_hbm.at[idx_v.at[pl.ds(row0 + j * _SUB, _SUB)]],
                    buf.at[pl.ds(j * _SUB, _SUB)],
                    sem))
            for c in copies:
                c.wait()

            def pe_body(s, c2):
                for d in range(D_MODEL // 16):
                    pev = pe_v[s, pl.ds(d * 16, 16)]
                    for rep in range(_CHUNK // SEQ_LEN):
                        r = rep * SEQ_LEN + s
                        buf[r, pl.ds(d * 16, 16)] = buf[r, pl.ds(d * 16, 16)] + pev
                return c2
            lax.fori_loop(0, SEQ_LEN, pe_body, 0)

            pltpu.sync_copy(buf, out_hbm.at[pl.ds(base + row0, _CHUNK)])
            return carry
        lax.fori_loop(0, _NCHUNK, chunk_body, 0)

    return k(table, idx_flat, pe)


def kernel(input_idx, table):
    idx_flat = input_idx.reshape(_TOTAL).astype(jnp.int32)
    out = _sc_embed(idx_flat, table, _pe_table())
    return out.reshape(BATCH, SEQ_LEN, D_MODEL)
